# initial kernel scaffold (unmeasured)
import jax
import jax.numpy as jnp
from jax import lax
from jax.experimental import pallas as pl
from jax.experimental.pallas import tpu as pltpu

B, SQ, H, D = 4, 32, 8, 128
SCALE = D ** -0.5
MESH = pl.DeviceIdType.MESH


def _partials(Q, K, V):
    skv = K.shape[1]

    def body(q_ref, k_ref, v_ref, o_ref, l_ref):
        q = q_ref[0, :, 0, :]
        k = k_ref[0, :, 0, :]
        v = v_ref[0, :, 0, :]
        s = lax.dot_general(q, k, (((1,), (1,)), ((), ())),
                            preferred_element_type=jnp.float32) * SCALE
        p = jnp.exp(s)
        l = jnp.sum(p, axis=1)
        o = lax.dot_general(p, v, (((1,), (0,)), ((), ())),
                            preferred_element_type=jnp.float32)
        o_ref[0, :, 0, :] = o
        l_ref[0] = l[:, None]

    return pl.pallas_call(
        body,
        grid=(B, H),
        in_specs=[
            pl.BlockSpec((1, SQ, 1, D), lambda b, h: (b, 0, h, 0)),
            pl.BlockSpec((1, skv, 1, D), lambda b, h: (b, 0, h, 0)),
            pl.BlockSpec((1, skv, 1, D), lambda b, h: (b, 0, h, 0)),
        ],
        out_specs=[
            pl.BlockSpec((1, SQ, 1), lambda b, h: (b, 0, h)),
            pl.BlockSpec((1, SQ, 1, D), lambda b, h: (b, 0, h, 0)),
        ],
        out_shape=[
            jax.ShapeDtypeStruct((B, SQ, H), jnp.float32),
            jax.ShapeDtypeStruct((B, SQ, H, D), jnp.float32),
        ],
    )(Q, K, V)


def _allreduce_combine(L, O_u):
    def body(o_ref, l_ref, out_ref,
             acc_o, acc_l, ro, rl, so, rso, sl, rsl):
        my_x = lax.axis_index("x")
        my_y = lax.axis_index("y")
        my_z = lax.axis_index("z")
        p1 = (my_x, my_y, my_z ^ 1)
        p2 = (my_x, my_y, my_z ^ 2)

        bar = pltpu.get_barrier_semaphore()
        for p in (p1, p2):
            pl.semaphore_signal(bar, inc=1, device_id=p, device_id_type=MESH)
        pl.semaphore_wait(bar, 2)

        r0o = pltpu.make_async_remote_copy(
            src_ref=o_ref, dst_ref=ro.at[0],
            send_sem=so.at[0], recv_sem=rso.at[0],
            device_id=p1, device_id_type=MESH)
        r0l = pltpu.make_async_remote_copy(
            src_ref=l_ref, dst_ref=rl.at[0],
            send_sem=sl.at[0], recv_sem=rsl.at[0],
            device_id=p1, device_id_type=MESH)
        r0o.start()
        r0l.start()
        r0o.wait()
        r0l.wait()
        acc_o[...] = o_ref[...] + ro[0]
        acc_l[...] = l_ref[...] + rl[0]

        r1o = pltpu.make_async_remote_copy(
            src_ref=acc_o, dst_ref=ro.at[1],
            send_sem=so.at[1], recv_sem=rso.at[1],
            device_id=p2, device_id_type=MESH)
        r1l = pltpu.make_async_remote_copy(
            src_ref=acc_l, dst_ref=rl.at[1],
            send_sem=sl.at[1], recv_sem=rsl.at[1],
            device_id=p2, device_id_type=MESH)
        r1o.start()
        r1l.start()
        r1o.wait()
        r1l.wait()

        num = acc_o[...] + ro[1]
        den = (acc_l[...] + rl[1])[..., None]
        out_ref[...] = num / den

    return pl.pallas_call(
        body,
        out_shape=jax.ShapeDtypeStruct((B, SQ, H, D), jnp.float32),
        in_specs=[
            pl.BlockSpec(memory_space=pltpu.VMEM),
            pl.BlockSpec(memory_space=pltpu.VMEM),
        ],
        out_specs=pl.BlockSpec(memory_space=pltpu.VMEM),
        scratch_shapes=[
            pltpu.VMEM((B, SQ, H, D), jnp.float32),
            pltpu.VMEM((B, SQ, H), jnp.float32),
            pltpu.VMEM((2, B, SQ, H, D), jnp.float32),
            pltpu.VMEM((2, B, SQ, H), jnp.float32),
            pltpu.SemaphoreType.DMA((2,)),
            pltpu.SemaphoreType.DMA((2,)),
            pltpu.SemaphoreType.DMA((2,)),
            pltpu.SemaphoreType.DMA((2,)),
        ],
        compiler_params=pltpu.CompilerParams(collective_id=0),
    )(O_u, L)


def kernel(Q, K, V):
    L, O_u = _partials(Q, K, V)
    return _allreduce_combine(L, O_u)


# baseline (device time: 108399 ns/iter reference)
import jax
import jax.numpy as jnp
from jax import lax
from jax.experimental import pallas as pl
from jax.experimental.pallas import tpu as pltpu

B, SQ, H, D = 4, 32, 8, 128
SCALE = D ** -0.5
MESH = pl.DeviceIdType.MESH


CHUNK = 1024


def _partials(Q, K, V):
    skv = K.shape[1]
    nc = skv // CHUNK

    def body(q_ref, k_ref, v_ref, l_ref, o_ref, o_acc, l_acc):
        c = pl.program_id(1)

        @pl.when(c == 0)
        def _():
            o_acc[...] = jnp.zeros_like(o_acc)
            l_acc[...] = jnp.zeros_like(l_acc)

        for h in range(H):
            q = q_ref[0, :, h, :]
            k = k_ref[0, :, h, :]
            v = v_ref[0, :, h, :]
            s = lax.dot_general(q, k, (((1,), (1,)), ((), ())),
                                preferred_element_type=jnp.float32) * SCALE
            p = jnp.exp(s)
            l_acc[:, h] += jnp.sum(p, axis=1)
            o_acc[:, h, :] += lax.dot_general(
                p, v, (((1,), (0,)), ((), ())),
                preferred_element_type=jnp.float32)

        @pl.when(c == nc - 1)
        def _():
            l_ref[0] = l_acc[...]
            o_ref[0] = o_acc[...]

    return pl.pallas_call(
        body,
        grid=(B, nc),
        in_specs=[
            pl.BlockSpec((1, SQ, H, D), lambda b, c: (b, 0, 0, 0)),
            pl.BlockSpec((1, CHUNK, H, D), lambda b, c: (b, c, 0, 0)),
            pl.BlockSpec((1, CHUNK, H, D), lambda b, c: (b, c, 0, 0)),
        ],
        out_specs=[
            pl.BlockSpec((1, SQ, H), lambda b, c: (b, 0, 0)),
            pl.BlockSpec((1, SQ, H, D), lambda b, c: (b, 0, 0, 0)),
        ],
        out_shape=[
            jax.ShapeDtypeStruct((B, SQ, H), jnp.float32),
            jax.ShapeDtypeStruct((B, SQ, H, D), jnp.float32),
        ],
        scratch_shapes=[
            pltpu.VMEM((SQ, H, D), jnp.float32),
            pltpu.VMEM((SQ, H), jnp.float32),
        ],
    )(Q, K, V)


def _allreduce_combine(L, O_u):
    def body(o_ref, l_ref, out_ref,
             acc_o, acc_l, ro, rl, so, rso, sl, rsl):
        my_x = lax.axis_index("x")
        my_y = lax.axis_index("y")
        my_z = lax.axis_index("z")
        p1 = (my_x, my_y, my_z ^ 1)
        p2 = (my_x, my_y, my_z ^ 2)

        bar = pltpu.get_barrier_semaphore()
        for p in (p1, p2):
            pl.semaphore_signal(bar, inc=1, device_id=p, device_id_type=MESH)
        pl.semaphore_wait(bar, 2)

        r0o = pltpu.make_async_remote_copy(
            src_ref=o_ref, dst_ref=ro.at[0],
            send_sem=so.at[0], recv_sem=rso.at[0],
            device_id=p1, device_id_type=MESH)
        r0l = pltpu.make_async_remote_copy(
            src_ref=l_ref, dst_ref=rl.at[0],
            send_sem=sl.at[0], recv_sem=rsl.at[0],
            device_id=p1, device_id_type=MESH)
        r0o.start()
        r0l.start()
        r0o.wait()
        r0l.wait()
        acc_o[...] = o_ref[...] + ro[0]
        acc_l[...] = l_ref[...] + rl[0]

        r1o = pltpu.make_async_remote_copy(
            src_ref=acc_o, dst_ref=ro.at[1],
            send_sem=so.at[1], recv_sem=rso.at[1],
            device_id=p2, device_id_type=MESH)
        r1l = pltpu.make_async_remote_copy(
            src_ref=acc_l, dst_ref=rl.at[1],
            send_sem=sl.at[1], recv_sem=rsl.at[1],
            device_id=p2, device_id_type=MESH)
        r1o.start()
        r1l.start()
        r1o.wait()
        r1l.wait()

        num = acc_o[...] + ro[1]
        den = (acc_l[...] + rl[1])[..., None]
        out_ref[...] = num / den

    return pl.pallas_call(
        body,
        out_shape=jax.ShapeDtypeStruct((B, SQ, H, D), jnp.float32),
        in_specs=[
            pl.BlockSpec(memory_space=pltpu.VMEM),
            pl.BlockSpec(memory_space=pltpu.VMEM),
        ],
        out_specs=pl.BlockSpec(memory_space=pltpu.VMEM),
        scratch_shapes=[
            pltpu.VMEM((B, SQ, H, D), jnp.float32),
            pltpu.VMEM((B, SQ, H), jnp.float32),
            pltpu.VMEM((2, B, SQ, H, D), jnp.float32),
            pltpu.VMEM((2, B, SQ, H), jnp.float32),
            pltpu.SemaphoreType.DMA((2,)),
            pltpu.SemaphoreType.DMA((2,)),
            pltpu.SemaphoreType.DMA((2,)),
            pltpu.SemaphoreType.DMA((2,)),
        ],
        compiler_params=pltpu.CompilerParams(collective_id=0),
    )(O_u, L)


def kernel(Q, K, V):
    L, O_u = _partials(Q, K, V)
    return _allreduce_combine(L, O_u)


# device time: 101153 ns/iter; 1.0716x vs baseline; 1.0716x over previous
import jax
import jax.numpy as jnp
from jax import lax
from jax.experimental import pallas as pl
from jax.experimental.pallas import tpu as pltpu

B, SQ, H, D = 4, 32, 8, 128
SCALE = D ** -0.5
CHUNK = 1024
MESHID = pl.DeviceIdType.MESH


def kernel(Q, K, V):
    skv = K.shape[1]
    nc = skv // CHUNK

    def body(q_ref, k_ref, v_ref, out_ref,
             o_acc, l_acc, o_part, l_part, acc_o, acc_l,
             ro0, rl0, ro1, rl1,
             s0o, r0o, s0l, r0l, s1o, r1o, s1l, r1l):
        b = pl.program_id(0)
        c = pl.program_id(1)
        my_x = lax.axis_index("x")
        my_y = lax.axis_index("y")
        my_z = lax.axis_index("z")
        p1 = (my_x, my_y, my_z ^ 1)
        p2 = (my_x, my_y, my_z ^ 2)

        def r0(slot):
            return pltpu.make_async_remote_copy(
                src_ref=o_part.at[slot], dst_ref=ro0.at[slot],
                send_sem=s0o.at[slot], recv_sem=r0o.at[slot],
                device_id=p1, device_id_type=MESHID)

        def r0_l(slot):
            return pltpu.make_async_remote_copy(
                src_ref=l_part.at[slot], dst_ref=rl0.at[slot],
                send_sem=s0l.at[slot], recv_sem=r0l.at[slot],
                device_id=p1, device_id_type=MESHID)

        def r1(slot):
            return pltpu.make_async_remote_copy(
                src_ref=acc_o.at[slot], dst_ref=ro1.at[slot],
                send_sem=s1o.at[slot], recv_sem=r1o.at[slot],
                device_id=p2, device_id_type=MESHID)

        def r1_l(slot):
            return pltpu.make_async_remote_copy(
                src_ref=acc_l.at[slot], dst_ref=rl1.at[slot],
                send_sem=s1l.at[slot], recv_sem=r1l.at[slot],
                device_id=p2, device_id_type=MESHID)

        @pl.when(jnp.logical_and(b == 0, c == 0))
        def _():
            bar = pltpu.get_barrier_semaphore()
            for p in (p1, p2):
                pl.semaphore_signal(bar, inc=1, device_id=p,
                                    device_id_type=MESHID)
            pl.semaphore_wait(bar, 2)

        @pl.when(c == 0)
        def _():
            o_acc[...] = jnp.zeros_like(o_acc)
            l_acc[...] = jnp.zeros_like(l_acc)

        for h in range(H):
            q = q_ref[0, :, h, :]
            k = k_ref[0, :, h, :]
            v = v_ref[0, :, h, :]
            s = lax.dot_general(q, k, (((1,), (1,)), ((), ())),
                                preferred_element_type=jnp.float32) * SCALE
            p = jnp.exp(s)
            l_acc[:, h] += jnp.sum(p, axis=1)
            o_acc[:, h, :] += lax.dot_general(
                p, v, (((1,), (0,)), ((), ())),
                preferred_element_type=jnp.float32)

        @pl.when(c == nc - 1)
        def _():
            o_part[b] = o_acc[...]
            l_part[b] = l_acc[...]
            r0(b).start()
            r0_l(b).start()

        @pl.when(jnp.logical_and(b == B - 1, c == nc - 1))
        def _():
            for bb in range(B):
                r0(bb).wait()
                r0_l(bb).wait()
                acc_o[bb] = o_part[bb] + ro0[bb]
                acc_l[bb] = l_part[bb] + rl0[bb]
                r1(bb).start()
                r1_l(bb).start()
            for bb in range(B):
                r1(bb).wait()
                r1_l(bb).wait()
                num = acc_o[bb] + ro1[bb]
                den = (acc_l[bb] + rl1[bb])[..., None]
                out_ref[bb] = num / den

    return pl.pallas_call(
        body,
        grid=(B, nc),
        in_specs=[
            pl.BlockSpec((1, SQ, H, D), lambda b, c: (b, 0, 0, 0)),
            pl.BlockSpec((1, CHUNK, H, D), lambda b, c: (b, c, 0, 0)),
            pl.BlockSpec((1, CHUNK, H, D), lambda b, c: (b, c, 0, 0)),
        ],
        out_specs=pl.BlockSpec((B, SQ, H, D), lambda b, c: (0, 0, 0, 0)),
        out_shape=jax.ShapeDtypeStruct((B, SQ, H, D), jnp.float32),
        scratch_shapes=[
            pltpu.VMEM((SQ, H, D), jnp.float32),
            pltpu.VMEM((SQ, H), jnp.float32),
            pltpu.VMEM((B, SQ, H, D), jnp.float32),
            pltpu.VMEM((B, SQ, H), jnp.float32),
            pltpu.VMEM((B, SQ, H, D), jnp.float32),
            pltpu.VMEM((B, SQ, H), jnp.float32),
            pltpu.VMEM((B, SQ, H, D), jnp.float32),
            pltpu.VMEM((B, SQ, H), jnp.float32),
            pltpu.VMEM((B, SQ, H, D), jnp.float32),
            pltpu.VMEM((B, SQ, H), jnp.float32),
            pltpu.SemaphoreType.DMA((B,)),
            pltpu.SemaphoreType.DMA((B,)),
            pltpu.SemaphoreType.DMA((B,)),
            pltpu.SemaphoreType.DMA((B,)),
            pltpu.SemaphoreType.DMA((B,)),
            pltpu.SemaphoreType.DMA((B,)),
            pltpu.SemaphoreType.DMA((B,)),
            pltpu.SemaphoreType.DMA((B,)),
        ],
        compiler_params=pltpu.CompilerParams(collective_id=0),
    )(Q, K, V)


# device time: 93990 ns/iter; 1.1533x vs baseline; 1.0762x over previous
import jax
import jax.numpy as jnp
from jax import lax
from jax.experimental import pallas as pl
from jax.experimental.pallas import tpu as pltpu

B, SQ, H, D = 4, 32, 8, 128
SCALE = D ** -0.5
CHUNK = 1024
MESHID = pl.DeviceIdType.MESH


def kernel(Q, K, V):
    skv = K.shape[1]
    nc = skv // CHUNK

    def body(q_ref, k_ref, v_ref, out_ref,
             o_acc, l_acc, o_part, l_part, acc_o, acc_l,
             ro0, rl0, ro1, rl1,
             s0o, r0o, s0l, r0l, s1o, r1o, s1l, r1l):
        b = pl.program_id(0)
        c = pl.program_id(1)
        my_x = lax.axis_index("x")
        my_y = lax.axis_index("y")
        my_z = lax.axis_index("z")
        p1 = (my_x, my_y, my_z ^ 1)
        p2 = (my_x, my_y, my_z ^ 2)

        def r0(slot):
            return pltpu.make_async_remote_copy(
                src_ref=o_part.at[slot], dst_ref=ro0.at[slot],
                send_sem=s0o.at[slot], recv_sem=r0o.at[slot],
                device_id=p1, device_id_type=MESHID)

        def r0_l(slot):
            return pltpu.make_async_remote_copy(
                src_ref=l_part.at[slot], dst_ref=rl0.at[slot],
                send_sem=s0l.at[slot], recv_sem=r0l.at[slot],
                device_id=p1, device_id_type=MESHID)

        def r1(slot):
            return pltpu.make_async_remote_copy(
                src_ref=acc_o.at[slot], dst_ref=ro1.at[slot],
                send_sem=s1o.at[slot], recv_sem=r1o.at[slot],
                device_id=p2, device_id_type=MESHID)

        def r1_l(slot):
            return pltpu.make_async_remote_copy(
                src_ref=acc_l.at[slot], dst_ref=rl1.at[slot],
                send_sem=s1l.at[slot], recv_sem=r1l.at[slot],
                device_id=p2, device_id_type=MESHID)

        @pl.when(jnp.logical_and(b == 0, c == 0))
        def _():
            bar = pltpu.get_barrier_semaphore()
            for p in (p1, p2):
                pl.semaphore_signal(bar, inc=1, device_id=p,
                                    device_id_type=MESHID)
            pl.semaphore_wait(bar, 2)

        @pl.when(c == 0)
        def _():
            o_acc[...] = jnp.zeros_like(o_acc)
            l_acc[...] = jnp.zeros_like(l_acc)

        for h in range(H):
            q = q_ref[0, :, h, :]
            k = k_ref[0, :, h, :]
            v = v_ref[0, :, h, :]
            s = lax.dot_general(q, k, (((1,), (1,)), ((), ())),
                                preferred_element_type=jnp.float32) * SCALE
            p = jnp.exp(s)
            l_acc[:, h] += jnp.sum(p, axis=1)
            o_acc[:, h, :] += lax.dot_general(
                p, v, (((1,), (0,)), ((), ())),
                preferred_element_type=jnp.float32)

        @pl.when(c == nc - 1)
        def _():
            o_part[b] = o_acc[...]
            l_part[b] = l_acc[...]
            r0(b).start()
            r0_l(b).start()

        @pl.when(jnp.logical_and(b >= 1, c == nc - 1))
        def _():
            bp = b - 1
            r0(bp).wait()
            r0_l(bp).wait()
            acc_o[bp] = o_part[bp] + ro0[bp]
            acc_l[bp] = l_part[bp] + rl0[bp]
            r1(bp).start()
            r1_l(bp).start()

        @pl.when(jnp.logical_and(b == B - 1, c == nc - 1))
        def _():
            bb = B - 1
            r0(bb).wait()
            r0_l(bb).wait()
            acc_o[bb] = o_part[bb] + ro0[bb]
            acc_l[bb] = l_part[bb] + rl0[bb]
            r1(bb).start()
            r1_l(bb).start()
            for bb in range(B):
                r1(bb).wait()
                r1_l(bb).wait()
                num = acc_o[bb] + ro1[bb]
                den = (acc_l[bb] + rl1[bb])[..., None]
                out_ref[bb] = num / den

    return pl.pallas_call(
        body,
        grid=(B, nc),
        in_specs=[
            pl.BlockSpec((1, SQ, H, D), lambda b, c: (b, 0, 0, 0)),
            pl.BlockSpec((1, CHUNK, H, D), lambda b, c: (b, c, 0, 0)),
            pl.BlockSpec((1, CHUNK, H, D), lambda b, c: (b, c, 0, 0)),
        ],
        out_specs=pl.BlockSpec((B, SQ, H, D), lambda b, c: (0, 0, 0, 0)),
        out_shape=jax.ShapeDtypeStruct((B, SQ, H, D), jnp.float32),
        scratch_shapes=[
            pltpu.VMEM((SQ, H, D), jnp.float32),
            pltpu.VMEM((SQ, H), jnp.float32),
            pltpu.VMEM((B, SQ, H, D), jnp.float32),
            pltpu.VMEM((B, SQ, H), jnp.float32),
            pltpu.VMEM((B, SQ, H, D), jnp.float32),
            pltpu.VMEM((B, SQ, H), jnp.float32),
            pltpu.VMEM((B, SQ, H, D), jnp.float32),
            pltpu.VMEM((B, SQ, H), jnp.float32),
            pltpu.VMEM((B, SQ, H, D), jnp.float32),
            pltpu.VMEM((B, SQ, H), jnp.float32),
            pltpu.SemaphoreType.DMA((B,)),
            pltpu.SemaphoreType.DMA((B,)),
            pltpu.SemaphoreType.DMA((B,)),
            pltpu.SemaphoreType.DMA((B,)),
            pltpu.SemaphoreType.DMA((B,)),
            pltpu.SemaphoreType.DMA((B,)),
            pltpu.SemaphoreType.DMA((B,)),
            pltpu.SemaphoreType.DMA((B,)),
        ],
        compiler_params=pltpu.CompilerParams(collective_id=0),
    )(Q, K, V)
